# Initial kernel scaffold; baseline (speedup 1.0000x reference)
import jax
import jax.numpy as jnp
from jax.experimental import pallas as pl


def kernel(pred_cls, pred_reg, anchor):
    raise NotImplementedError("write your pallas kernel here")


# speedup vs baseline: 68.4603x; 68.4603x over previous
import jax
import jax.numpy as jnp
from jax.experimental import pallas as pl
from jax.experimental.pallas import tpu as pltpu

_MIN_SIZE = 16.0 / 800.0
_IOU_THR = 0.7
_PRE_NMS = 6000
_POST_NMS = 300
_NP = 19968
_KP = 6016
_B = 128
_NB = _KP // _B
_R = 304


def _decode_kernel(c_ref, t_ref, a_ref, s_ref, roi_ref):
    c0 = c_ref[0:1, :]
    c1 = c_ref[1:2, :]
    ax1 = a_ref[0:1, :]
    ay1 = a_ref[1:2, :]
    ax2 = a_ref[2:3, :]
    ay2 = a_ref[3:4, :]
    acx = (ax2 + ax1) / 2.0
    acy = (ay2 + ay1) / 2.0
    aw = ax2 - ax1
    ah = ay2 - ay1
    tx = t_ref[0:1, :]
    ty = t_ref[1:2, :]
    tw = t_ref[2:3, :]
    th = t_ref[3:4, :]
    cx = tx * aw + acx
    cy = ty * ah + acy
    w = jnp.exp(tw) * aw
    h = jnp.exp(th) * ah
    x1 = jnp.clip(cx - w / 2.0, 0.0, 1.0)
    y1 = jnp.clip(cy - h / 2.0, 0.0, 1.0)
    x2 = jnp.clip(cx + w / 2.0, 0.0, 1.0)
    y2 = jnp.clip(cy + h / 2.0, 0.0, 1.0)
    ws = x2 - x1
    hs = y2 - y1
    ok = (hs >= _MIN_SIZE) & (ws >= _MIN_SIZE)
    m = jnp.maximum(c0, c1)
    e0 = jnp.exp(c0 - m)
    e1 = jnp.exp(c1 - m)
    s = e1 / (e0 + e1)
    s_ref[...] = jnp.where(ok, s, -jnp.inf)
    roi_ref[0:1, :] = x1
    roi_ref[1:2, :] = y1
    roi_ref[2:3, :] = x2
    roi_ref[3:4, :] = y2


def _iou(x1c, y1c, x2c, y2c, areac, x1r, y1r, x2r, y2r, arear):
    ix1 = jnp.maximum(x1c, x1r)
    iy1 = jnp.maximum(y1c, y1r)
    ix2 = jnp.minimum(x2c, x2r)
    iy2 = jnp.minimum(y2c, y2r)
    inter = jnp.maximum(ix2 - ix1, 0.0) * jnp.maximum(iy2 - iy1, 0.0)
    union = areac + arear - inter
    return inter / (union + 1e-9)


def _nms_kernel(bx_ref, bx3_ref, bt3_ref, bt_ref, fin_ref, out_ref, keep_ref):
    x1r = bt_ref[0:1, :]
    y1r = bt_ref[1:2, :]
    x2r = bt_ref[2:3, :]
    y2r = bt_ref[3:4, :]
    arear = jnp.maximum(x2r - x1r, 0.0) * jnp.maximum(y2r - y1r, 0.0)
    colg = jax.lax.broadcasted_iota(jnp.int32, (1, _KP), 1)

    for b2 in range(_NB):
        keep_ref[b2] = jnp.ones((1, _B), jnp.float32)

    def block_body(b, carry):
        base = b * _B
        blk = bx3_ref[b]
        x1c = blk[:, 0:1]
        y1c = blk[:, 1:2]
        x2c = blk[:, 2:3]
        y2c = blk[:, 3:4]
        areac = jnp.maximum(x2c - x1c, 0.0) * jnp.maximum(y2c - y1c, 0.0)
        blkt = bt3_ref[b]
        x1rb = blkt[0:1, :]
        y1rb = blkt[1:2, :]
        x2rb = blkt[2:3, :]
        y2rb = blkt[3:4, :]
        arearb = (jnp.maximum(x2rb - x1rb, 0.0)
                  * jnp.maximum(y2rb - y1rb, 0.0))
        rowg = base + jax.lax.broadcasted_iota(jnp.int32, (_B, 1), 0)

        iou_all = _iou(x1c, y1c, x2c, y2c, areac, x1r, y1r, x2r, y2r, arear)
        adj = ((iou_all > _IOU_THR) & (colg > rowg)).astype(jnp.float32)

        colb = base + jax.lax.broadcasted_iota(jnp.int32, (1, _B), 1)
        iou_bb = _iou(x1c, y1c, x2c, y2c, areac,
                      x1rb, y1rb, x2rb, y2rb, arearb)
        adj_bb = ((iou_bb > _IOU_THR) & (colb > rowg)).astype(jnp.float32)

        alive = keep_ref[b]

        def w_cond(st):
            return st[1]

        def w_body(st):
            k, _ = st
            sup = jnp.dot(k, adj_bb, preferred_element_type=jnp.float32)
            knew = jnp.where(sup > 0.0, 0.0, alive)
            return knew, jnp.any(knew != k)

        kfin, _ = jax.lax.while_loop(w_cond, w_body, (alive, jnp.asarray(True)))
        keep_ref[b] = kfin
        sup_all = jnp.dot(kfin, adj, preferred_element_type=jnp.float32)
        for b2 in range(_NB):
            part = sup_all[0:1, b2 * _B:(b2 + 1) * _B]
            later = jnp.asarray(b2, jnp.int32) > b
            keep_ref[b2] = jnp.where(later & (part > 0.0), 0.0, keep_ref[b2])
        return carry

    jax.lax.fori_loop(0, _NB, block_body, 0, unroll=False)

    keep_row = jnp.concatenate([keep_ref[b2] for b2 in range(_NB)], axis=1)
    sel = keep_row * fin_ref[...]
    ltri = (jax.lax.broadcasted_iota(jnp.int32, (_B, _B), 0)
            <= jax.lax.broadcasted_iota(jnp.int32, (_B, _B), 1)).astype(jnp.float32)
    parts = []
    carry = jnp.zeros((1, 1), jnp.float32)
    for b2 in range(_NB):
        selb = sel[0:1, b2 * _B:(b2 + 1) * _B]
        local = jnp.dot(selb, ltri, preferred_element_type=jnp.float32) + carry
        parts.append(local)
        carry = carry + jnp.sum(selb)
    cum = jnp.concatenate(parts, axis=1)
    rows = (jax.lax.broadcasted_iota(jnp.int32, (_R, _KP), 0) + 1).astype(jnp.float32)
    e = ((cum == rows) & (sel > 0.0)).astype(jnp.float32)
    out_ref[...] = jnp.dot(e, bx_ref[...], preferred_element_type=jnp.float32)


def kernel(pred_cls, pred_reg, anchor):
    n = pred_reg.shape[1]
    c01 = jnp.pad(pred_cls.reshape(n, 2).T, ((0, 0), (0, _NP - n)))
    regt = jnp.pad(pred_reg[0].T, ((0, 0), (0, _NP - n)))
    ancht = jnp.pad(anchor.T, ((0, 0), (0, _NP - n)))

    scores_row, roit = pl.pallas_call(
        _decode_kernel,
        out_shape=(
            jax.ShapeDtypeStruct((1, _NP), jnp.float32),
            jax.ShapeDtypeStruct((4, _NP), jnp.float32),
        ),
    )(c01, regt, ancht)

    scores = scores_row[0, :n]
    vals, idx = jax.lax.top_k(scores, _PRE_NMS)
    boxes = roit[:, :n].T[idx]
    boxes = jnp.pad(boxes, ((0, _KP - _PRE_NMS), (0, 0)))
    bx3 = boxes.reshape(_NB, _B, 4)
    bt3 = jnp.transpose(bx3, (0, 2, 1))
    boxest = boxes.T
    finite = jnp.pad(jnp.isfinite(vals).astype(jnp.float32),
                     (0, _KP - _PRE_NMS)).reshape(1, _KP)

    out = pl.pallas_call(
        _nms_kernel,
        out_shape=jax.ShapeDtypeStruct((_R, 4), jnp.float32),
        scratch_shapes=[pltpu.VMEM((_NB, 1, _B), jnp.float32)],
    )(boxes, bx3, bt3, boxest, finite)
    return out[:_POST_NMS]
